# Initial kernel scaffold; baseline (speedup 1.0000x reference)
import jax
import jax.numpy as jnp
from jax.experimental import pallas as pl


def kernel(x, h1, c1, x1, del_t, f, iou1, mso1, W_q, b_q, W_k, b_k, W_c, b_c, U_f, U_iou, U_mso, b_iou, b_mso, b, a, edge_src):
    raise NotImplementedError("write your pallas kernel here")


# speedup vs baseline: 3.4388x; 3.4388x over previous
import functools

import jax
import jax.numpy as jnp
from jax import lax
from jax.experimental import pallas as pl
from jax.experimental.pallas import tpu as pltpu
from jax.experimental.pallas import tpu_sc as plsc

_EPS = 1e-15
_MAXN = 1.0 - 1e-5

_D = 128
_DEG = 16
_TW = 7 * _D

_NC = 2
_NS = 16
_NW = _NC * _NS
_CH = 64

_BP = 512
_BM = 64


def _norm(x):
    return jnp.sqrt(jnp.sum(x * x, axis=-1, keepdims=True) + _EPS)


def _project(x):
    n = _norm(x)
    return jnp.where(n > _MAXN, x / n * _MAXN, x)


def _artanh(x):
    x = jnp.clip(x, -1.0 + 1e-7, 1.0 - 1e-7)
    return 0.5 * jnp.log((1.0 + x) / (1.0 - x))


def _expmap0(u):
    n = jnp.maximum(_norm(u), _EPS)
    return _project(jnp.tanh(n) * u / n)


def _logmap0(x):
    n = jnp.maximum(_norm(x), _EPS)
    return _artanh(n) * x / n


def _mobius_add(x, y):
    x2 = jnp.sum(x * x, -1, keepdims=True)
    y2 = jnp.sum(y * y, -1, keepdims=True)
    xy = jnp.sum(x * y, -1, keepdims=True)
    num = (1.0 + 2.0 * xy + y2) * x + (1.0 - x2) * y
    den = 1.0 + 2.0 * xy + x2 * y2
    return _project(num / jnp.maximum(den, _EPS))


def _mobius_matvec(M, x):
    mx = lax.dot_general(x, M, (((x.ndim - 1,), (1,)), ((), ())))
    xn = jnp.maximum(_norm(x), _EPS)
    mxn = jnp.maximum(_norm(mx), _EPS)
    return _project(jnp.tanh(mxn / xn * _artanh(xn)) * mx / mxn)


def _mobius_pointwise_mul(w, x):
    wx = w * x
    xn = jnp.maximum(_norm(x), _EPS)
    wxn = jnp.maximum(_norm(wx), _EPS)
    return _project(jnp.tanh(wxn / xn * _artanh(xn)) * wx / wxn)


def _mobius_scalar_mul(r, x):
    n = jnp.maximum(_norm(x), _EPS)
    return _project(jnp.tanh(r * _artanh(n)) * x / n)


def _weighted_midpoint(xs):
    lam = 2.0 / jnp.maximum(1.0 - jnp.sum(xs * xs, -1, keepdims=True), _EPS)
    nom = jnp.sum(lam * xs, axis=1)
    den = jnp.maximum(jnp.sum(lam - 1.0, axis=1), _EPS)
    return _mobius_scalar_mul(0.5, nom / den)


def _prep_body(x_ref, h1_ref, c1_ref, x1_ref, dt_ref, b_ref, a_ref,
               wq_ref, bq_ref, wk_ref, bk_ref, wc_ref, bc_ref, uf_ref,
               table_ref, xq_ref):
    h1p = _project(h1_ref[...])
    c1p = _project(c1_ref[...])
    xq = _mobius_add(_mobius_matvec(wq_ref[...], _expmap0(x_ref[...])), bq_ref[...])
    hk = _mobius_add(_mobius_matvec(wk_ref[...], h1p), bk_ref[...])
    c_sk = _expmap0(jnp.tanh(_logmap0(
        _mobius_add(_mobius_matvec(wc_ref[...], c1p), bc_ref[...]))))
    tmp = _mobius_matvec(uf_ref[...], h1p)
    g = b_ref[0, 0] * jnp.exp(-a_ref[0, 0] * dt_ref[...])
    table_ref[:, 0 * _D:1 * _D] = h1p
    table_ref[:, 1 * _D:2 * _D] = c1p
    table_ref[:, 2 * _D:3 * _D] = x1_ref[...]
    table_ref[:, 3 * _D:4 * _D] = hk
    table_ref[:, 4 * _D:5 * _D] = c_sk
    table_ref[:, 5 * _D:6 * _D] = tmp
    table_ref[:, 6 * _D:7 * _D] = jnp.broadcast_to(g, (g.shape[0], _D))
    xq_ref[...] = xq


def _prep(xp, h1p_, c1p_, x1p_, dtp, b2, a2, W_q, b_q, W_k, b_k, W_c, b_c, U_f):
    npad = xp.shape[0]
    row = lambda i: (i, 0)
    rep = lambda i: (0, 0)
    return pl.pallas_call(
        _prep_body,
        grid=(npad // _BP,),
        in_specs=[
            pl.BlockSpec((_BP, _D), row),
            pl.BlockSpec((_BP, _D), row),
            pl.BlockSpec((_BP, _D), row),
            pl.BlockSpec((_BP, _D), row),
            pl.BlockSpec((_BP, 1), row),
            pl.BlockSpec((1, 1), rep),
            pl.BlockSpec((1, 1), rep),
            pl.BlockSpec((_D, _D), rep),
            pl.BlockSpec((1, _D), rep),
            pl.BlockSpec((_D, _D), rep),
            pl.BlockSpec((1, _D), rep),
            pl.BlockSpec((_D, _D), rep),
            pl.BlockSpec((1, _D), rep),
            pl.BlockSpec((_D, _D), rep),
        ],
        out_specs=[
            pl.BlockSpec((_BP, _TW), row),
            pl.BlockSpec((_BP, _D), row),
        ],
        out_shape=[
            jax.ShapeDtypeStruct((npad, _TW), jnp.float32),
            jax.ShapeDtypeStruct((npad, _D), jnp.float32),
        ],
    )(xp, h1p_, c1p_, x1p_, dtp, b2, a2, W_q, b_q, W_k, b_k, W_c, b_c, U_f)


def _sc_gather(table, idx3):
    nchunk = idx3.shape[1]
    e_pad = _NW * nchunk * _CH
    mesh = plsc.VectorSubcoreMesh(core_axis_name="c", subcore_axis_name="s")

    @functools.partial(
        pl.kernel, mesh=mesh,
        out_type=jax.ShapeDtypeStruct((e_pad, _TW), jnp.float32),
        scratch_types=[
            pltpu.VMEM((nchunk, _CH), jnp.int32),
            pltpu.VMEM((_CH, _TW), jnp.float32),
            pltpu.SemaphoreType.DMA,
        ],
    )
    def k(table_hbm, idx_hbm, out_hbm, idx_v, buf, sem):
        wid = lax.axis_index("s") * _NC + lax.axis_index("c")
        base = wid * (nchunk * _CH)
        pltpu.sync_copy(idx_hbm.at[wid], idx_v)

        def body(c, carry):
            pltpu.async_copy(table_hbm.at[idx_v.at[c]], buf, sem).wait()
            pltpu.sync_copy(buf, out_hbm.at[pl.ds(base + c * _CH, _CH)])
            return carry

        lax.fori_loop(0, nchunk, body, 0)

    return k(table, idx3)


def _main_body(mail_ref, x_ref, xq_ref, f_ref, iou1_ref, mso1_ref,
               uiou_ref, umso_ref, biou_ref, bmso_ref,
               h_out_ref, c_out_ref, x_out_ref):
    h_m = mail_ref[:, :, 0 * _D:1 * _D]
    c_m = mail_ref[:, :, 1 * _D:2 * _D]
    x_m = mail_ref[:, :, 2 * _D:3 * _D]
    hk_m = mail_ref[:, :, 3 * _D:4 * _D]
    csk_m = mail_ref[:, :, 4 * _D:5 * _D]
    tmp_m = mail_ref[:, :, 5 * _D:6 * _D]
    g_t = mail_ref[:, :, 6 * _D:6 * _D + 1]

    xq = xq_ref[...][:, None, :]
    d = 2.0 * _artanh(_norm(_mobius_add(-xq, hk_m)))
    scores = jax.nn.softmax(-d, axis=1)
    scaled = scores * g_t
    h_tild = _weighted_midpoint(_mobius_pointwise_mul(scaled, h_m))

    c_sk_hat = _mobius_pointwise_mul(csk_m, g_t)
    c_k_tilde = _mobius_add(_mobius_add(-csk_m, c_m), c_sk_hat)
    f_p = _project(f_ref[...])[:, None, :]
    fg = jax.nn.sigmoid(_logmap0(
        _mobius_add(jnp.broadcast_to(f_p, tmp_m.shape), tmp_m)))
    c_red = _weighted_midpoint(_mobius_pointwise_mul(fg, c_k_tilde))
    x_red = jnp.mean(x_m, axis=1)

    iou1n = _mobius_add(_project(iou1_ref[...]), _mobius_matvec(uiou_ref[...], h_tild))
    mso1n = _mobius_add(_project(mso1_ref[...]), _mobius_matvec(umso_ref[...], h_tild))
    iou = _mobius_add(iou1n, biou_ref[...])
    mso = _mobius_add(mso1n, bmso_ref[...])
    i_ = jax.nn.sigmoid(_logmap0(iou[:, 0 * _D:1 * _D]))
    u_ = jnp.tanh(_logmap0(iou[:, 1 * _D:2 * _D]))
    m_ = jax.nn.sigmoid(_logmap0(mso[:, 0 * _D:1 * _D]))
    s_ = jax.nn.sigmoid(_logmap0(mso[:, 1 * _D:2 * _D]))
    o_ = jax.nn.sigmoid(_logmap0(mso[:, 2 * _D:3 * _D]))
    c_out = _mobius_add(
        _mobius_add(_mobius_pointwise_mul(i_, u_), c_red),
        _mobius_pointwise_mul(m_, s_))
    h_out_ref[...] = _mobius_pointwise_mul(o_, jnp.tanh(_logmap0(c_out)))
    c_out_ref[...] = c_out
    x_out_ref[...] = (x_red + x_ref[...]) * 0.5


def _main(mail3, xp, x_q, fp, iou1p, mso1p, U_iou, U_mso, b_iou, b_mso):
    npad = xp.shape[0]
    row = lambda i: (i, 0)
    rep = lambda i: (0, 0)
    return pl.pallas_call(
        _main_body,
        grid=(npad // _BM,),
        in_specs=[
            pl.BlockSpec((_BM, _DEG, _TW), lambda i: (i, 0, 0)),
            pl.BlockSpec((_BM, _D), row),
            pl.BlockSpec((_BM, _D), row),
            pl.BlockSpec((_BM, _D), row),
            pl.BlockSpec((_BM, 2 * _D), row),
            pl.BlockSpec((_BM, 3 * _D), row),
            pl.BlockSpec((2 * _D, _D), rep),
            pl.BlockSpec((3 * _D, _D), rep),
            pl.BlockSpec((1, 2 * _D), rep),
            pl.BlockSpec((1, 3 * _D), rep),
        ],
        out_specs=[
            pl.BlockSpec((_BM, _D), row),
            pl.BlockSpec((_BM, _D), row),
            pl.BlockSpec((_BM, _D), row),
        ],
        out_shape=[
            jax.ShapeDtypeStruct((npad, _D), jnp.float32),
            jax.ShapeDtypeStruct((npad, _D), jnp.float32),
            jax.ShapeDtypeStruct((npad, _D), jnp.float32),
        ],
    )(mail3, xp, x_q, fp, iou1p, mso1p, U_iou, U_mso, b_iou, b_mso)


def kernel(x, h1, c1, x1, del_t, f, iou1, mso1, W_q, b_q, W_k, b_k, W_c, b_c,
           U_f, U_iou, U_mso, b_iou, b_mso, b, a, edge_src):
    n, d = x.shape
    deg = edge_src.shape[1]
    npad = ((n + _BP - 1) // _BP) * _BP
    pad = npad - n

    pad2 = lambda t: jnp.pad(t, ((0, pad), (0, 0)))
    xp = pad2(x)
    dtp = jnp.pad(del_t, (0, pad)).reshape(npad, 1)
    b2 = b.reshape(1, 1)
    a2 = a.reshape(1, 1)

    table, x_q = _prep(xp, pad2(h1), pad2(c1), pad2(x1), dtp, b2, a2,
                       W_q, b_q, W_k, b_k, W_c, b_c, U_f)

    nchunk = (npad * deg) // (_NW * _CH)
    idx3 = jnp.pad(edge_src.reshape(-1), (0, pad * deg)).reshape(_NW, nchunk, _CH)
    mail = _sc_gather(table, idx3)
    mail3 = mail.reshape(npad, deg, _TW)

    h_out, c_out, x_out = _main(mail3, xp, x_q, pad2(f), pad2(iou1), pad2(mso1),
                                U_iou, U_mso, b_iou, b_mso)
    return h_out[:n], c_out[:n], x_out[:n]


# speedup vs baseline: 3.5450x; 1.0309x over previous
import functools

import jax
import jax.numpy as jnp
from jax import lax
from jax.experimental import pallas as pl
from jax.experimental.pallas import tpu as pltpu
from jax.experimental.pallas import tpu_sc as plsc

_EPS = 1e-15
_MAXN = 1.0 - 1e-5

_D = 128
_DEG = 16
_TW = 7 * _D

_NC = 2
_NS = 16
_NW = _NC * _NS
_CH = 64

_BP = 512
_BM = 64


def _norm(x):
    return jnp.sqrt(jnp.sum(x * x, axis=-1, keepdims=True) + _EPS)


def _project(x):
    n = _norm(x)
    return jnp.where(n > _MAXN, x / n * _MAXN, x)


def _artanh(x):
    x = jnp.clip(x, -1.0 + 1e-7, 1.0 - 1e-7)
    return 0.5 * jnp.log((1.0 + x) / (1.0 - x))


def _expmap0(u):
    n = jnp.maximum(_norm(u), _EPS)
    return _project(jnp.tanh(n) * u / n)


def _logmap0(x):
    n = jnp.maximum(_norm(x), _EPS)
    return _artanh(n) * x / n


def _mobius_add(x, y):
    x2 = jnp.sum(x * x, -1, keepdims=True)
    y2 = jnp.sum(y * y, -1, keepdims=True)
    xy = jnp.sum(x * y, -1, keepdims=True)
    num = (1.0 + 2.0 * xy + y2) * x + (1.0 - x2) * y
    den = 1.0 + 2.0 * xy + x2 * y2
    return _project(num / jnp.maximum(den, _EPS))


def _mobius_matvec(M, x):
    mx = lax.dot_general(x, M, (((x.ndim - 1,), (1,)), ((), ())))
    xn = jnp.maximum(_norm(x), _EPS)
    mxn = jnp.maximum(_norm(mx), _EPS)
    return _project(jnp.tanh(mxn / xn * _artanh(xn)) * mx / mxn)


def _mobius_pointwise_mul(w, x):
    wx = w * x
    xn = jnp.maximum(_norm(x), _EPS)
    wxn = jnp.maximum(_norm(wx), _EPS)
    return _project(jnp.tanh(wxn / xn * _artanh(xn)) * wx / wxn)


def _mobius_scalar_mul(r, x):
    n = jnp.maximum(_norm(x), _EPS)
    return _project(jnp.tanh(r * _artanh(n)) * x / n)


def _weighted_midpoint(xs):
    lam = 2.0 / jnp.maximum(1.0 - jnp.sum(xs * xs, -1, keepdims=True), _EPS)
    nom = jnp.sum(lam * xs, axis=1)
    den = jnp.maximum(jnp.sum(lam - 1.0, axis=1), _EPS)
    return _mobius_scalar_mul(0.5, nom / den)


def _prep_body(x_ref, h1_ref, c1_ref, x1_ref, dt_ref, b_ref, a_ref,
               wq_ref, bq_ref, wk_ref, bk_ref, wc_ref, bc_ref, uf_ref,
               table_ref, xq_ref):
    h1p = _project(h1_ref[...])
    c1p = _project(c1_ref[...])
    xq = _mobius_add(_mobius_matvec(wq_ref[...], _expmap0(x_ref[...])), bq_ref[...])
    hk = _mobius_add(_mobius_matvec(wk_ref[...], h1p), bk_ref[...])
    c_sk = _expmap0(jnp.tanh(_logmap0(
        _mobius_add(_mobius_matvec(wc_ref[...], c1p), bc_ref[...]))))
    tmp = _mobius_matvec(uf_ref[...], h1p)
    g = b_ref[0, 0] * jnp.exp(-a_ref[0, 0] * dt_ref[...])
    table_ref[:, 0 * _D:1 * _D] = h1p
    table_ref[:, 1 * _D:2 * _D] = c1p
    table_ref[:, 2 * _D:3 * _D] = x1_ref[...]
    table_ref[:, 3 * _D:4 * _D] = hk
    table_ref[:, 4 * _D:5 * _D] = c_sk
    table_ref[:, 5 * _D:6 * _D] = tmp
    table_ref[:, 6 * _D:7 * _D] = jnp.broadcast_to(g, (g.shape[0], _D))
    xq_ref[...] = xq


def _prep(xp, h1p_, c1p_, x1p_, dtp, b2, a2, W_q, b_q, W_k, b_k, W_c, b_c, U_f):
    npad = xp.shape[0]
    row = lambda i: (i, 0)
    rep = lambda i: (0, 0)
    return pl.pallas_call(
        _prep_body,
        grid=(npad // _BP,),
        in_specs=[
            pl.BlockSpec((_BP, _D), row),
            pl.BlockSpec((_BP, _D), row),
            pl.BlockSpec((_BP, _D), row),
            pl.BlockSpec((_BP, _D), row),
            pl.BlockSpec((_BP, 1), row),
            pl.BlockSpec((1, 1), rep),
            pl.BlockSpec((1, 1), rep),
            pl.BlockSpec((_D, _D), rep),
            pl.BlockSpec((1, _D), rep),
            pl.BlockSpec((_D, _D), rep),
            pl.BlockSpec((1, _D), rep),
            pl.BlockSpec((_D, _D), rep),
            pl.BlockSpec((1, _D), rep),
            pl.BlockSpec((_D, _D), rep),
        ],
        out_specs=[
            pl.BlockSpec((_BP, _TW), row),
            pl.BlockSpec((_BP, _D), row),
        ],
        out_shape=[
            jax.ShapeDtypeStruct((npad, _TW), jnp.float32),
            jax.ShapeDtypeStruct((npad, _D), jnp.float32),
        ],
    )(xp, h1p_, c1p_, x1p_, dtp, b2, a2, W_q, b_q, W_k, b_k, W_c, b_c, U_f)


def _sc_gather(table, idx3):
    nchunk = idx3.shape[1]
    e_pad = _NW * nchunk * _CH
    mesh = plsc.VectorSubcoreMesh(core_axis_name="c", subcore_axis_name="s")

    @functools.partial(
        pl.kernel, mesh=mesh,
        out_type=jax.ShapeDtypeStruct((e_pad, _TW), jnp.float32),
        scratch_types=[
            pltpu.VMEM((nchunk, _CH), jnp.int32),
            pltpu.VMEM((_CH, _TW), jnp.float32),
            pltpu.VMEM((_CH, _TW), jnp.float32),
            pltpu.SemaphoreType.DMA,
            pltpu.SemaphoreType.DMA,
            pltpu.SemaphoreType.DMA,
            pltpu.SemaphoreType.DMA,
        ],
    )
    def k(table_hbm, idx_hbm, out_hbm, idx_v, buf0, buf1, gs0, gs1, ws0, ws1):
        wid = lax.axis_index("s") * _NC + lax.axis_index("c")
        base = wid * (nchunk * _CH)
        pltpu.sync_copy(idx_hbm.at[wid], idx_v)

        bufs = (buf0, buf1)
        gsem = (gs0, gs1)
        wsem = (ws0, ws1)

        def g_start(cc, p):
            pltpu.async_copy(table_hbm.at[idx_v.at[cc]], bufs[p], gsem[p])

        def g_wait(p):
            pltpu.make_async_copy(
                table_hbm.at[idx_v.at[0]], bufs[p], gsem[p]).wait()

        def w_start(cc, p):
            pltpu.async_copy(
                bufs[p], out_hbm.at[pl.ds(base + cc * _CH, _CH)], wsem[p])

        def w_wait(p):
            pltpu.make_async_copy(
                bufs[p], out_hbm.at[pl.ds(base, _CH)], wsem[p]).wait()

        g_start(0, 0)
        g_wait(0)
        w_start(0, 0)
        g_start(1, 1)

        def body(i, carry):
            for b in range(2):
                cc = 1 + 2 * i + b
                p = (1 + b) % 2
                g_wait(p)
                w_start(cc, p)
                w_wait(1 - p)
                g_start(cc + 1, 1 - p)
            return carry

        lax.fori_loop(0, (nchunk - 2) // 2, body, 0)

        pl1 = (nchunk - 1) % 2
        g_wait(pl1)
        w_start(nchunk - 1, pl1)
        w_wait(1 - pl1)
        w_wait(pl1)

    return k(table, idx3)


def _main_body(mail_ref, x_ref, xq_ref, f_ref, iou1_ref, mso1_ref,
               uiou_ref, umso_ref, biou_ref, bmso_ref,
               h_out_ref, c_out_ref, x_out_ref):
    h_m = mail_ref[:, :, 0 * _D:1 * _D]
    c_m = mail_ref[:, :, 1 * _D:2 * _D]
    x_m = mail_ref[:, :, 2 * _D:3 * _D]
    hk_m = mail_ref[:, :, 3 * _D:4 * _D]
    csk_m = mail_ref[:, :, 4 * _D:5 * _D]
    tmp_m = mail_ref[:, :, 5 * _D:6 * _D]
    g_t = mail_ref[:, :, 6 * _D:6 * _D + 1]

    xq = xq_ref[...][:, None, :]
    d = 2.0 * _artanh(_norm(_mobius_add(-xq, hk_m)))
    scores = jax.nn.softmax(-d, axis=1)
    scaled = scores * g_t
    h_tild = _weighted_midpoint(_mobius_pointwise_mul(scaled, h_m))

    c_sk_hat = _mobius_pointwise_mul(csk_m, g_t)
    c_k_tilde = _mobius_add(_mobius_add(-csk_m, c_m), c_sk_hat)
    f_p = _project(f_ref[...])[:, None, :]
    fg = jax.nn.sigmoid(_logmap0(
        _mobius_add(jnp.broadcast_to(f_p, tmp_m.shape), tmp_m)))
    c_red = _weighted_midpoint(_mobius_pointwise_mul(fg, c_k_tilde))
    x_red = jnp.mean(x_m, axis=1)

    iou1n = _mobius_add(_project(iou1_ref[...]), _mobius_matvec(uiou_ref[...], h_tild))
    mso1n = _mobius_add(_project(mso1_ref[...]), _mobius_matvec(umso_ref[...], h_tild))
    iou = _mobius_add(iou1n, biou_ref[...])
    mso = _mobius_add(mso1n, bmso_ref[...])
    i_ = jax.nn.sigmoid(_logmap0(iou[:, 0 * _D:1 * _D]))
    u_ = jnp.tanh(_logmap0(iou[:, 1 * _D:2 * _D]))
    m_ = jax.nn.sigmoid(_logmap0(mso[:, 0 * _D:1 * _D]))
    s_ = jax.nn.sigmoid(_logmap0(mso[:, 1 * _D:2 * _D]))
    o_ = jax.nn.sigmoid(_logmap0(mso[:, 2 * _D:3 * _D]))
    c_out = _mobius_add(
        _mobius_add(_mobius_pointwise_mul(i_, u_), c_red),
        _mobius_pointwise_mul(m_, s_))
    h_out_ref[...] = _mobius_pointwise_mul(o_, jnp.tanh(_logmap0(c_out)))
    c_out_ref[...] = c_out
    x_out_ref[...] = (x_red + x_ref[...]) * 0.5


def _main(mail3, xp, x_q, fp, iou1p, mso1p, U_iou, U_mso, b_iou, b_mso):
    npad = xp.shape[0]
    row = lambda i: (i, 0)
    rep = lambda i: (0, 0)
    return pl.pallas_call(
        _main_body,
        grid=(npad // _BM,),
        in_specs=[
            pl.BlockSpec((_BM, _DEG, _TW), lambda i: (i, 0, 0)),
            pl.BlockSpec((_BM, _D), row),
            pl.BlockSpec((_BM, _D), row),
            pl.BlockSpec((_BM, _D), row),
            pl.BlockSpec((_BM, 2 * _D), row),
            pl.BlockSpec((_BM, 3 * _D), row),
            pl.BlockSpec((2 * _D, _D), rep),
            pl.BlockSpec((3 * _D, _D), rep),
            pl.BlockSpec((1, 2 * _D), rep),
            pl.BlockSpec((1, 3 * _D), rep),
        ],
        out_specs=[
            pl.BlockSpec((_BM, _D), row),
            pl.BlockSpec((_BM, _D), row),
            pl.BlockSpec((_BM, _D), row),
        ],
        out_shape=[
            jax.ShapeDtypeStruct((npad, _D), jnp.float32),
            jax.ShapeDtypeStruct((npad, _D), jnp.float32),
            jax.ShapeDtypeStruct((npad, _D), jnp.float32),
        ],
    )(mail3, xp, x_q, fp, iou1p, mso1p, U_iou, U_mso, b_iou, b_mso)


def kernel(x, h1, c1, x1, del_t, f, iou1, mso1, W_q, b_q, W_k, b_k, W_c, b_c,
           U_f, U_iou, U_mso, b_iou, b_mso, b, a, edge_src):
    n, d = x.shape
    deg = edge_src.shape[1]
    npad = ((n + _BP - 1) // _BP) * _BP
    pad = npad - n

    pad2 = lambda t: jnp.pad(t, ((0, pad), (0, 0)))
    xp = pad2(x)
    dtp = jnp.pad(del_t, (0, pad)).reshape(npad, 1)
    b2 = b.reshape(1, 1)
    a2 = a.reshape(1, 1)

    table, x_q = _prep(xp, pad2(h1), pad2(c1), pad2(x1), dtp, b2, a2,
                       W_q, b_q, W_k, b_k, W_c, b_c, U_f)

    nchunk = (npad * deg) // (_NW * _CH)
    idx3 = jnp.pad(edge_src.reshape(-1), (0, pad * deg)).reshape(_NW, nchunk, _CH)
    mail = _sc_gather(table, idx3)
    mail3 = mail.reshape(npad, deg, _TW)

    h_out, c_out, x_out = _main(mail3, xp, x_q, pad2(f), pad2(iou1), pad2(mso1),
                                U_iou, U_mso, b_iou, b_mso)
    return h_out[:n], c_out[:n], x_out[:n]


# speedup vs baseline: 4.6357x; 1.3077x over previous
import functools

import jax
import jax.numpy as jnp
from jax import lax
from jax.experimental import pallas as pl
from jax.experimental.pallas import tpu as pltpu
from jax.experimental.pallas import tpu_sc as plsc

_EPS = 1e-15
_MAXN = 1.0 - 1e-5

_D = 128
_DEG = 16
_TW = 5 * _D

_NC = 2
_NS = 16
_NW = _NC * _NS
_CH = 64

_BP = 512
_BM = 64


def _norm(x):
    return jnp.sqrt(jnp.sum(x * x, axis=-1, keepdims=True) + _EPS)


def _project(x):
    n = _norm(x)
    return jnp.where(n > _MAXN, x / n * _MAXN, x)


def _artanh(x):
    x = jnp.clip(x, -1.0 + 1e-7, 1.0 - 1e-7)
    return 0.5 * jnp.log((1.0 + x) / (1.0 - x))


def _expmap0(u):
    n = jnp.maximum(_norm(u), _EPS)
    return _project(jnp.tanh(n) * u / n)


def _logmap0(x):
    n = jnp.maximum(_norm(x), _EPS)
    return _artanh(n) * x / n


def _mobius_add(x, y):
    x2 = jnp.sum(x * x, -1, keepdims=True)
    y2 = jnp.sum(y * y, -1, keepdims=True)
    xy = jnp.sum(x * y, -1, keepdims=True)
    num = (1.0 + 2.0 * xy + y2) * x + (1.0 - x2) * y
    den = 1.0 + 2.0 * xy + x2 * y2
    return _project(num / jnp.maximum(den, _EPS))


def _mobius_matvec(M, x):
    mx = lax.dot_general(x, M, (((x.ndim - 1,), (1,)), ((), ())))
    xn = jnp.maximum(_norm(x), _EPS)
    mxn = jnp.maximum(_norm(mx), _EPS)
    return _project(jnp.tanh(mxn / xn * _artanh(xn)) * mx / mxn)


def _mobius_pointwise_mul(w, x):
    wx = w * x
    xn = jnp.maximum(_norm(x), _EPS)
    wxn = jnp.maximum(_norm(wx), _EPS)
    return _project(jnp.tanh(wxn / xn * _artanh(xn)) * wx / wxn)


def _mobius_scalar_mul(r, x):
    n = jnp.maximum(_norm(x), _EPS)
    return _project(jnp.tanh(r * _artanh(n)) * x / n)


def _weighted_midpoint(xs):
    lam = 2.0 / jnp.maximum(1.0 - jnp.sum(xs * xs, -1, keepdims=True), _EPS)
    nom = jnp.sum(lam * xs, axis=1)
    den = jnp.maximum(jnp.sum(lam - 1.0, axis=1), _EPS)
    return _mobius_scalar_mul(0.5, nom / den)


def _prep_body(x_ref, h1_ref, c1_ref, x1_ref, dt_ref, b_ref, a_ref,
               wq_ref, bq_ref, wk_ref, bk_ref, wc_ref, bc_ref, uf_ref,
               table_ref, xq_ref):
    h1p = _project(h1_ref[...])
    c1p = _project(c1_ref[...])
    xq = _mobius_add(_mobius_matvec(wq_ref[...], _expmap0(x_ref[...])), bq_ref[...])
    hk = _mobius_add(_mobius_matvec(wk_ref[...], h1p), bk_ref[...])
    c_sk = _expmap0(jnp.tanh(_logmap0(
        _mobius_add(_mobius_matvec(wc_ref[...], c1p), bc_ref[...]))))
    tmp = _mobius_matvec(uf_ref[...], h1p)
    g = b_ref[0, 0] * jnp.exp(-a_ref[0, 0] * dt_ref[...])
    csk_hat = _mobius_pointwise_mul(c_sk, g)
    ckt = _mobius_add(_mobius_add(-c_sk, c1p), csk_hat)
    table_ref[:, 0 * _D:1 * _D] = g * _logmap0(h1p)
    table_ref[:, 1 * _D:2 * _D] = x1_ref[...]
    table_ref[:, 2 * _D:3 * _D] = hk
    table_ref[:, 3 * _D:4 * _D] = ckt
    table_ref[:, 4 * _D:5 * _D] = tmp
    xq_ref[...] = xq


def _prep(xp, h1p_, c1p_, x1p_, dtp, b2, a2, W_q, b_q, W_k, b_k, W_c, b_c, U_f):
    npad = xp.shape[0]
    row = lambda i: (i, 0)
    rep = lambda i: (0, 0)
    return pl.pallas_call(
        _prep_body,
        grid=(npad // _BP,),
        in_specs=[
            pl.BlockSpec((_BP, _D), row),
            pl.BlockSpec((_BP, _D), row),
            pl.BlockSpec((_BP, _D), row),
            pl.BlockSpec((_BP, _D), row),
            pl.BlockSpec((_BP, 1), row),
            pl.BlockSpec((1, 1), rep),
            pl.BlockSpec((1, 1), rep),
            pl.BlockSpec((_D, _D), rep),
            pl.BlockSpec((1, _D), rep),
            pl.BlockSpec((_D, _D), rep),
            pl.BlockSpec((1, _D), rep),
            pl.BlockSpec((_D, _D), rep),
            pl.BlockSpec((1, _D), rep),
            pl.BlockSpec((_D, _D), rep),
        ],
        out_specs=[
            pl.BlockSpec((_BP, _TW), row),
            pl.BlockSpec((_BP, _D), row),
        ],
        out_shape=[
            jax.ShapeDtypeStruct((npad, _TW), jnp.float32),
            jax.ShapeDtypeStruct((npad, _D), jnp.float32),
        ],
    )(xp, h1p_, c1p_, x1p_, dtp, b2, a2, W_q, b_q, W_k, b_k, W_c, b_c, U_f)


def _sc_gather(table, idx3):
    nchunk = idx3.shape[1]
    e_pad = _NW * nchunk * _CH
    mesh = plsc.VectorSubcoreMesh(core_axis_name="c", subcore_axis_name="s")

    @functools.partial(
        pl.kernel, mesh=mesh,
        out_type=jax.ShapeDtypeStruct((e_pad, _TW), jnp.float32),
        scratch_types=[
            pltpu.VMEM((nchunk, _CH), jnp.int32),
            pltpu.VMEM((_CH, _TW), jnp.float32),
            pltpu.VMEM((_CH, _TW), jnp.float32),
            pltpu.SemaphoreType.DMA,
            pltpu.SemaphoreType.DMA,
            pltpu.SemaphoreType.DMA,
            pltpu.SemaphoreType.DMA,
        ],
    )
    def k(table_hbm, idx_hbm, out_hbm, idx_v, buf0, buf1, gs0, gs1, ws0, ws1):
        wid = lax.axis_index("s") * _NC + lax.axis_index("c")
        base = wid * (nchunk * _CH)
        pltpu.sync_copy(idx_hbm.at[wid], idx_v)

        bufs = (buf0, buf1)
        gsem = (gs0, gs1)
        wsem = (ws0, ws1)

        def g_start(cc, p):
            pltpu.async_copy(table_hbm.at[idx_v.at[cc]], bufs[p], gsem[p])

        def g_wait(p):
            pltpu.make_async_copy(
                table_hbm.at[idx_v.at[0]], bufs[p], gsem[p]).wait()

        def w_start(cc, p):
            pltpu.async_copy(
                bufs[p], out_hbm.at[pl.ds(base + cc * _CH, _CH)], wsem[p])

        def w_wait(p):
            pltpu.make_async_copy(
                bufs[p], out_hbm.at[pl.ds(base, _CH)], wsem[p]).wait()

        g_start(0, 0)
        g_wait(0)
        w_start(0, 0)
        g_start(1, 1)

        def body(i, carry):
            for b in range(2):
                cc = 1 + 2 * i + b
                p = (1 + b) % 2
                g_wait(p)
                w_start(cc, p)
                w_wait(1 - p)
                g_start(cc + 1, 1 - p)
            return carry

        lax.fori_loop(0, (nchunk - 2) // 2, body, 0)

        pl1 = (nchunk - 1) % 2
        g_wait(pl1)
        w_start(nchunk - 1, pl1)
        w_wait(1 - pl1)
        w_wait(pl1)

    return k(table, idx3)


def _main_body(mail_ref, x_ref, xq_ref, f_ref, iou1_ref, mso1_ref,
               uiou_ref, umso_ref, biou_ref, bmso_ref,
               h_out_ref, c_out_ref, x_out_ref):
    glh_m = mail_ref[:, :, 0 * _D:1 * _D]
    x_m = mail_ref[:, :, 1 * _D:2 * _D]
    hk_m = mail_ref[:, :, 2 * _D:3 * _D]
    ckt_m = mail_ref[:, :, 3 * _D:4 * _D]
    tmp_m = mail_ref[:, :, 4 * _D:5 * _D]

    xq = xq_ref[...][:, None, :]
    d = 2.0 * _artanh(_norm(_mobius_add(-xq, hk_m)))
    scores = jax.nn.softmax(-d, axis=1)
    h_tild = _weighted_midpoint(_expmap0(scores * glh_m))

    f_p = _project(f_ref[...])[:, None, :]
    fg = jax.nn.sigmoid(_logmap0(
        _mobius_add(jnp.broadcast_to(f_p, tmp_m.shape), tmp_m)))
    c_red = _weighted_midpoint(_mobius_pointwise_mul(fg, ckt_m))
    x_red = jnp.mean(x_m, axis=1)

    iou1n = _mobius_add(_project(iou1_ref[...]), _mobius_matvec(uiou_ref[...], h_tild))
    mso1n = _mobius_add(_project(mso1_ref[...]), _mobius_matvec(umso_ref[...], h_tild))
    iou = _mobius_add(iou1n, biou_ref[...])
    mso = _mobius_add(mso1n, bmso_ref[...])
    i_ = jax.nn.sigmoid(_logmap0(iou[:, 0 * _D:1 * _D]))
    u_ = jnp.tanh(_logmap0(iou[:, 1 * _D:2 * _D]))
    m_ = jax.nn.sigmoid(_logmap0(mso[:, 0 * _D:1 * _D]))
    s_ = jax.nn.sigmoid(_logmap0(mso[:, 1 * _D:2 * _D]))
    o_ = jax.nn.sigmoid(_logmap0(mso[:, 2 * _D:3 * _D]))
    c_out = _mobius_add(
        _mobius_add(_mobius_pointwise_mul(i_, u_), c_red),
        _mobius_pointwise_mul(m_, s_))
    h_out_ref[...] = _mobius_pointwise_mul(o_, jnp.tanh(_logmap0(c_out)))
    c_out_ref[...] = c_out
    x_out_ref[...] = (x_red + x_ref[...]) * 0.5


def _main(mail3, xp, x_q, fp, iou1p, mso1p, U_iou, U_mso, b_iou, b_mso):
    npad = xp.shape[0]
    row = lambda i: (i, 0)
    rep = lambda i: (0, 0)
    return pl.pallas_call(
        _main_body,
        grid=(npad // _BM,),
        in_specs=[
            pl.BlockSpec((_BM, _DEG, _TW), lambda i: (i, 0, 0)),
            pl.BlockSpec((_BM, _D), row),
            pl.BlockSpec((_BM, _D), row),
            pl.BlockSpec((_BM, _D), row),
            pl.BlockSpec((_BM, 2 * _D), row),
            pl.BlockSpec((_BM, 3 * _D), row),
            pl.BlockSpec((2 * _D, _D), rep),
            pl.BlockSpec((3 * _D, _D), rep),
            pl.BlockSpec((1, 2 * _D), rep),
            pl.BlockSpec((1, 3 * _D), rep),
        ],
        out_specs=[
            pl.BlockSpec((_BM, _D), row),
            pl.BlockSpec((_BM, _D), row),
            pl.BlockSpec((_BM, _D), row),
        ],
        out_shape=[
            jax.ShapeDtypeStruct((npad, _D), jnp.float32),
            jax.ShapeDtypeStruct((npad, _D), jnp.float32),
            jax.ShapeDtypeStruct((npad, _D), jnp.float32),
        ],
    )(mail3, xp, x_q, fp, iou1p, mso1p, U_iou, U_mso, b_iou, b_mso)


def kernel(x, h1, c1, x1, del_t, f, iou1, mso1, W_q, b_q, W_k, b_k, W_c, b_c,
           U_f, U_iou, U_mso, b_iou, b_mso, b, a, edge_src):
    n, d = x.shape
    deg = edge_src.shape[1]
    npad = ((n + _BP - 1) // _BP) * _BP
    pad = npad - n

    pad2 = lambda t: jnp.pad(t, ((0, pad), (0, 0)))
    xp = pad2(x)
    dtp = jnp.pad(del_t, (0, pad)).reshape(npad, 1)
    b2 = b.reshape(1, 1)
    a2 = a.reshape(1, 1)

    table, x_q = _prep(xp, pad2(h1), pad2(c1), pad2(x1), dtp, b2, a2,
                       W_q, b_q, W_k, b_k, W_c, b_c, U_f)

    nchunk = (npad * deg) // (_NW * _CH)
    idx3 = jnp.pad(edge_src.reshape(-1), (0, pad * deg)).reshape(_NW, nchunk, _CH)
    mail = _sc_gather(table, idx3)
    mail3 = mail.reshape(npad, deg, _TW)

    h_out, c_out, x_out = _main(mail3, xp, x_q, pad2(f), pad2(iou1), pad2(mso1),
                                U_iou, U_mso, b_iou, b_mso)
    return h_out[:n], c_out[:n], x_out[:n]


# speedup vs baseline: 6.1873x; 1.3347x over previous
import functools

import jax
import jax.numpy as jnp
from jax import lax
from jax.experimental import pallas as pl
from jax.experimental.pallas import tpu as pltpu
from jax.experimental.pallas import tpu_sc as plsc

_EPS = 1e-15
_MAXN = 1.0 - 1e-5

_D = 128
_DEG = 16
_TW = 5 * _D

_NC = 2
_NS = 16
_NW = _NC * _NS
_CH = 64

_BP = 512
_BM = 64


def _norm(x):
    return jnp.sqrt(jnp.sum(x * x, axis=-1, keepdims=True) + _EPS)


def _project(x):
    n = _norm(x)
    return jnp.where(n > _MAXN, x / n * _MAXN, x)


def _artanh(x):
    x = jnp.clip(x, -1.0 + 1e-7, 1.0 - 1e-7)
    return 0.5 * jnp.log((1.0 + x) / (1.0 - x))


def _expmap0(u):
    n = jnp.maximum(_norm(u), _EPS)
    return _project(jnp.tanh(n) * u / n)


def _logmap0(x):
    n = jnp.maximum(_norm(x), _EPS)
    return _artanh(n) * x / n


def _mobius_add(x, y):
    x2 = jnp.sum(x * x, -1, keepdims=True)
    y2 = jnp.sum(y * y, -1, keepdims=True)
    xy = jnp.sum(x * y, -1, keepdims=True)
    num = (1.0 + 2.0 * xy + y2) * x + (1.0 - x2) * y
    den = 1.0 + 2.0 * xy + x2 * y2
    return _project(num / jnp.maximum(den, _EPS))


def _mobius_matvec(M, x):
    mx = lax.dot_general(x, M, (((x.ndim - 1,), (1,)), ((), ())))
    xn = jnp.maximum(_norm(x), _EPS)
    mxn = jnp.maximum(_norm(mx), _EPS)
    return _project(jnp.tanh(mxn / xn * _artanh(xn)) * mx / mxn)


def _mobius_pointwise_mul(w, x):
    wx = w * x
    xn = jnp.maximum(_norm(x), _EPS)
    wxn = jnp.maximum(_norm(wx), _EPS)
    return _project(jnp.tanh(wxn / xn * _artanh(xn)) * wx / wxn)


def _mobius_scalar_mul(r, x):
    n = jnp.maximum(_norm(x), _EPS)
    return _project(jnp.tanh(r * _artanh(n)) * x / n)


def _weighted_midpoint(xs):
    lam = 2.0 / jnp.maximum(1.0 - jnp.sum(xs * xs, -1, keepdims=True), _EPS)
    nom = jnp.sum(lam * xs, axis=1)
    den = jnp.maximum(jnp.sum(lam - 1.0, axis=1), _EPS)
    return _mobius_scalar_mul(0.5, nom / den)


def _prep_body(x_ref, h1_ref, c1_ref, x1_ref, dt_ref, b_ref, a_ref,
               wq_ref, bq_ref, wk_ref, bk_ref, wc_ref, bc_ref, uf_ref,
               table_ref, xq_ref):
    h1p = _project(h1_ref[...])
    c1p = _project(c1_ref[...])
    xq = _mobius_add(_mobius_matvec(wq_ref[...], _expmap0(x_ref[...])), bq_ref[...])
    hk = _mobius_add(_mobius_matvec(wk_ref[...], h1p), bk_ref[...])
    c_sk = _expmap0(jnp.tanh(_logmap0(
        _mobius_add(_mobius_matvec(wc_ref[...], c1p), bc_ref[...]))))
    tmp = _mobius_matvec(uf_ref[...], h1p)
    g = b_ref[0, 0] * jnp.exp(-a_ref[0, 0] * dt_ref[...])
    csk_hat = _mobius_pointwise_mul(c_sk, g)
    ckt = _mobius_add(_mobius_add(-c_sk, c1p), csk_hat)
    table_ref[:, 0 * _D:1 * _D] = g * _logmap0(h1p)
    table_ref[:, 1 * _D:2 * _D] = x1_ref[...]
    table_ref[:, 2 * _D:3 * _D] = hk
    table_ref[:, 3 * _D:4 * _D] = ckt
    table_ref[:, 4 * _D:5 * _D] = tmp
    xq_ref[...] = xq


def _prep(xp, h1p_, c1p_, x1p_, dtp, b2, a2, W_q, b_q, W_k, b_k, W_c, b_c, U_f):
    npad = xp.shape[0]
    row = lambda i: (i, 0)
    rep = lambda i: (0, 0)
    return pl.pallas_call(
        _prep_body,
        grid=(npad // _BP,),
        in_specs=[
            pl.BlockSpec((_BP, _D), row),
            pl.BlockSpec((_BP, _D), row),
            pl.BlockSpec((_BP, _D), row),
            pl.BlockSpec((_BP, _D), row),
            pl.BlockSpec((_BP, 1), row),
            pl.BlockSpec((1, 1), rep),
            pl.BlockSpec((1, 1), rep),
            pl.BlockSpec((_D, _D), rep),
            pl.BlockSpec((1, _D), rep),
            pl.BlockSpec((_D, _D), rep),
            pl.BlockSpec((1, _D), rep),
            pl.BlockSpec((_D, _D), rep),
            pl.BlockSpec((1, _D), rep),
            pl.BlockSpec((_D, _D), rep),
        ],
        out_specs=[
            pl.BlockSpec((_BP, _TW), row),
            pl.BlockSpec((_BP, _D), row),
        ],
        out_shape=[
            jax.ShapeDtypeStruct((npad, _TW), jnp.float32),
            jax.ShapeDtypeStruct((npad, _D), jnp.float32),
        ],
    )(xp, h1p_, c1p_, x1p_, dtp, b2, a2, W_q, b_q, W_k, b_k, W_c, b_c, U_f)


def _sc_gather(table, idx3):
    nchunk = idx3.shape[1]
    e_pad = _NW * nchunk * _CH
    mesh = plsc.VectorSubcoreMesh(core_axis_name="c", subcore_axis_name="s")

    @functools.partial(
        pl.kernel, mesh=mesh,
        out_type=jax.ShapeDtypeStruct((e_pad, _TW), jnp.float32),
        scratch_types=[
            pltpu.VMEM((nchunk, _CH), jnp.int32),
            pltpu.VMEM((_CH, _TW), jnp.float32),
            pltpu.VMEM((_CH, _TW), jnp.float32),
            pltpu.SemaphoreType.DMA,
            pltpu.SemaphoreType.DMA,
            pltpu.SemaphoreType.DMA,
            pltpu.SemaphoreType.DMA,
        ],
    )
    def k(table_hbm, idx_hbm, out_hbm, idx_v, buf0, buf1, gs0, gs1, ws0, ws1):
        wid = lax.axis_index("s") * _NC + lax.axis_index("c")
        base = wid * (nchunk * _CH)
        pltpu.sync_copy(idx_hbm.at[wid], idx_v)

        bufs = (buf0, buf1)
        gsem = (gs0, gs1)
        wsem = (ws0, ws1)

        def g_start(cc, p):
            pltpu.async_copy(table_hbm.at[idx_v.at[cc]], bufs[p], gsem[p])

        def g_wait(p):
            pltpu.make_async_copy(
                table_hbm.at[idx_v.at[0]], bufs[p], gsem[p]).wait()

        def w_start(cc, p):
            pltpu.async_copy(
                bufs[p], out_hbm.at[pl.ds(base + cc * _CH, _CH)], wsem[p])

        def w_wait(p):
            pltpu.make_async_copy(
                bufs[p], out_hbm.at[pl.ds(base, _CH)], wsem[p]).wait()

        g_start(0, 0)
        g_wait(0)
        w_start(0, 0)
        g_start(1, 1)

        def body(i, carry):
            for b in range(2):
                cc = 1 + 2 * i + b
                p = (1 + b) % 2
                g_wait(p)
                w_start(cc, p)
                w_wait(1 - p)
                g_start(cc + 1, 1 - p)
            return carry

        lax.fori_loop(0, (nchunk - 2) // 2, body, 0)

        pl1 = (nchunk - 1) % 2
        g_wait(pl1)
        w_start(nchunk - 1, pl1)
        w_wait(1 - pl1)
        w_wait(pl1)

    return k(table, idx3)


def _main_body(mail_ref, x_ref, xq_ref, f_ref, iou1_ref, mso1_ref,
               uiou_ref, umso_ref, biou_ref, bmso_ref,
               h_out_ref, c_out_ref, x_out_ref):
    glh_m = mail_ref[:, :, 0 * _D:1 * _D]
    x_m = mail_ref[:, :, 1 * _D:2 * _D]
    hk_m = mail_ref[:, :, 2 * _D:3 * _D]
    ckt_m = mail_ref[:, :, 3 * _D:4 * _D]
    tmp_m = mail_ref[:, :, 4 * _D:5 * _D]

    xq = xq_ref[...][:, None, :]
    d = 2.0 * _artanh(_norm(_mobius_add(-xq, hk_m)))
    scores = jax.nn.softmax(-d, axis=1)
    h_tild = _weighted_midpoint(_expmap0(scores * glh_m))

    f_p = _project(f_ref[...])[:, None, :]
    fg = jax.nn.sigmoid(_logmap0(
        _mobius_add(jnp.broadcast_to(f_p, tmp_m.shape), tmp_m)))
    c_red = _weighted_midpoint(_mobius_pointwise_mul(fg, ckt_m))
    x_red = jnp.mean(x_m, axis=1)

    iou1n = _mobius_add(_project(iou1_ref[...]), _mobius_matvec(uiou_ref[...], h_tild))
    mso1n = _mobius_add(_project(mso1_ref[...]), _mobius_matvec(umso_ref[...], h_tild))
    iou = _mobius_add(iou1n, biou_ref[...])
    mso = _mobius_add(mso1n, bmso_ref[...])
    i_ = jax.nn.sigmoid(_logmap0(iou[:, 0 * _D:1 * _D]))
    u_ = jnp.tanh(_logmap0(iou[:, 1 * _D:2 * _D]))
    m_ = jax.nn.sigmoid(_logmap0(mso[:, 0 * _D:1 * _D]))
    s_ = jax.nn.sigmoid(_logmap0(mso[:, 1 * _D:2 * _D]))
    o_ = jax.nn.sigmoid(_logmap0(mso[:, 2 * _D:3 * _D]))
    c_out = _mobius_add(
        _mobius_add(_mobius_pointwise_mul(i_, u_), c_red),
        _mobius_pointwise_mul(m_, s_))
    h_out_ref[...] = _mobius_pointwise_mul(o_, jnp.tanh(_logmap0(c_out)))
    c_out_ref[...] = c_out
    x_out_ref[...] = (x_red + x_ref[...]) * 0.5


def _main(mail3, xp, x_q, fp, iou1p, mso1p, U_iou, U_mso, b_iou, b_mso):
    npad = xp.shape[0]
    row = lambda i: (i, 0)
    rep = lambda i: (0, 0)
    return pl.pallas_call(
        _main_body,
        grid=(npad // _BM,),
        in_specs=[
            pl.BlockSpec((_BM, _DEG, _TW), lambda i: (i, 0, 0)),
            pl.BlockSpec((_BM, _D), row),
            pl.BlockSpec((_BM, _D), row),
            pl.BlockSpec((_BM, _D), row),
            pl.BlockSpec((_BM, 2 * _D), row),
            pl.BlockSpec((_BM, 3 * _D), row),
            pl.BlockSpec((2 * _D, _D), rep),
            pl.BlockSpec((3 * _D, _D), rep),
            pl.BlockSpec((1, 2 * _D), rep),
            pl.BlockSpec((1, 3 * _D), rep),
        ],
        out_specs=[
            pl.BlockSpec((_BM, _D), row),
            pl.BlockSpec((_BM, _D), row),
            pl.BlockSpec((_BM, _D), row),
        ],
        out_shape=[
            jax.ShapeDtypeStruct((npad, _D), jnp.float32),
            jax.ShapeDtypeStruct((npad, _D), jnp.float32),
            jax.ShapeDtypeStruct((npad, _D), jnp.float32),
        ],
    )(mail3, xp, x_q, fp, iou1p, mso1p, U_iou, U_mso, b_iou, b_mso)


_NSLICE = 4


def kernel(x, h1, c1, x1, del_t, f, iou1, mso1, W_q, b_q, W_k, b_k, W_c, b_c,
           U_f, U_iou, U_mso, b_iou, b_mso, b, a, edge_src):
    n, d = x.shape
    deg = edge_src.shape[1]
    npad = ((n + _BP * _NSLICE - 1) // (_BP * _NSLICE)) * (_BP * _NSLICE)
    pad = npad - n

    pad2 = lambda t: jnp.pad(t, ((0, pad), (0, 0)))
    xp = pad2(x)
    dtp = jnp.pad(del_t, (0, pad)).reshape(npad, 1)
    b2 = b.reshape(1, 1)
    a2 = a.reshape(1, 1)

    table, x_q = _prep(xp, pad2(h1), pad2(c1), pad2(x1), dtp, b2, a2,
                       W_q, b_q, W_k, b_k, W_c, b_c, U_f)

    ns = npad // _NSLICE
    nchunk = (ns * deg) // (_NW * _CH)
    idx5 = jnp.pad(edge_src.reshape(-1), (0, pad * deg)).reshape(
        _NSLICE, _NW, nchunk, _CH)
    fp, iou1p, mso1p = pad2(f), pad2(iou1), pad2(mso1)

    houts, couts, xouts = [], [], []
    for s in range(_NSLICE):
        mail3 = _sc_gather(table, idx5[s]).reshape(ns, deg, _TW)
        lo, hi = s * ns, (s + 1) * ns
        h_o, c_o, x_o = _main(mail3, xp[lo:hi], x_q[lo:hi], fp[lo:hi],
                              iou1p[lo:hi], mso1p[lo:hi],
                              U_iou, U_mso, b_iou, b_mso)
        houts.append(h_o)
        couts.append(c_o)
        xouts.append(x_o)
    h_out = jnp.concatenate(houts)
    c_out = jnp.concatenate(couts)
    x_out = jnp.concatenate(xouts)
    return h_out[:n], c_out[:n], x_out[:n]


# speedup vs baseline: 6.1976x; 1.0017x over previous
import functools
import math

import jax
import jax.numpy as jnp
from jax import lax
from jax.experimental import pallas as pl
from jax.experimental.pallas import tpu as pltpu
from jax.experimental.pallas import tpu_sc as plsc

_EPS = 1e-15
_MAXN = 1.0 - 1e-5

_D = 128
_DEG = 16
_TW = 5 * _D

_NC = 2
_NS = 16
_NW = _NC * _NS
_CH = 64

_BP = 512
_BM = 64


def _norm(x):
    return jnp.sqrt(jnp.sum(x * x, axis=-1, keepdims=True) + _EPS)


def _project(x):
    n = _norm(x)
    return jnp.where(n > _MAXN, x / n * _MAXN, x)


def _artanh(x):
    x = jnp.clip(x, -1.0 + 1e-7, 1.0 - 1e-7)
    return 0.5 * jnp.log((1.0 + x) / (1.0 - x))


def _expmap0(u):
    n = jnp.maximum(_norm(u), _EPS)
    return _project(jnp.tanh(n) * u / n)


def _logmap0(x):
    n = jnp.maximum(_norm(x), _EPS)
    return _artanh(n) * x / n


def _mobius_add(x, y):
    x2 = jnp.sum(x * x, -1, keepdims=True)
    y2 = jnp.sum(y * y, -1, keepdims=True)
    xy = jnp.sum(x * y, -1, keepdims=True)
    num = (1.0 + 2.0 * xy + y2) * x + (1.0 - x2) * y
    den = 1.0 + 2.0 * xy + x2 * y2
    return _project(num / jnp.maximum(den, _EPS))


def _mobius_matvec(M, x):
    mx = lax.dot_general(x, M, (((x.ndim - 1,), (1,)), ((), ())))
    xn = jnp.maximum(_norm(x), _EPS)
    mxn = jnp.maximum(_norm(mx), _EPS)
    return _project(jnp.tanh(mxn / xn * _artanh(xn)) * mx / mxn)


def _mobius_pointwise_mul(w, x):
    wx = w * x
    xn = jnp.maximum(_norm(x), _EPS)
    wxn = jnp.maximum(_norm(wx), _EPS)
    return _project(jnp.tanh(wxn / xn * _artanh(xn)) * wx / wxn)


def _mobius_scalar_mul(r, x):
    n = jnp.maximum(_norm(x), _EPS)
    return _project(jnp.tanh(r * _artanh(n)) * x / n)


def _weighted_midpoint(xs):
    lam = 2.0 / jnp.maximum(1.0 - jnp.sum(xs * xs, -1, keepdims=True), _EPS)
    nom = jnp.sum(lam * xs, axis=1)
    den = jnp.maximum(jnp.sum(lam - 1.0, axis=1), _EPS)
    return _mobius_scalar_mul(0.5, nom / den)


def _prep_body(x_ref, h1_ref, c1_ref, x1_ref, dt_ref, b_ref, a_ref,
               wq_ref, bq_ref, wk_ref, bk_ref, wc_ref, bc_ref, uf_ref,
               table_ref, xq_ref):
    h1p = _project(h1_ref[...])
    c1p = _project(c1_ref[...])
    xq = _mobius_add(_mobius_matvec(wq_ref[...], _expmap0(x_ref[...])), bq_ref[...])
    hk = _mobius_add(_mobius_matvec(wk_ref[...], h1p), bk_ref[...])
    c_sk = _expmap0(jnp.tanh(_logmap0(
        _mobius_add(_mobius_matvec(wc_ref[...], c1p), bc_ref[...]))))
    tmp = _mobius_matvec(uf_ref[...], h1p)
    g = b_ref[0, 0] * jnp.exp(-a_ref[0, 0] * dt_ref[...])
    csk_hat = _mobius_pointwise_mul(c_sk, g)
    ckt = _mobius_add(_mobius_add(-c_sk, c1p), csk_hat)
    table_ref[:, 0 * _D:1 * _D] = g * _logmap0(h1p)
    table_ref[:, 1 * _D:2 * _D] = x1_ref[...]
    table_ref[:, 2 * _D:3 * _D] = hk
    table_ref[:, 3 * _D:4 * _D] = ckt
    table_ref[:, 4 * _D:5 * _D] = tmp
    xq_ref[...] = xq


def _prep(xp, h1p_, c1p_, x1p_, dtp, b2, a2, W_q, b_q, W_k, b_k, W_c, b_c, U_f):
    npad = xp.shape[0]
    row = lambda i: (i, 0)
    rep = lambda i: (0, 0)
    return pl.pallas_call(
        _prep_body,
        grid=(npad // _BP,),
        in_specs=[
            pl.BlockSpec((_BP, _D), row),
            pl.BlockSpec((_BP, _D), row),
            pl.BlockSpec((_BP, _D), row),
            pl.BlockSpec((_BP, _D), row),
            pl.BlockSpec((_BP, 1), row),
            pl.BlockSpec((1, 1), rep),
            pl.BlockSpec((1, 1), rep),
            pl.BlockSpec((_D, _D), rep),
            pl.BlockSpec((1, _D), rep),
            pl.BlockSpec((_D, _D), rep),
            pl.BlockSpec((1, _D), rep),
            pl.BlockSpec((_D, _D), rep),
            pl.BlockSpec((1, _D), rep),
            pl.BlockSpec((_D, _D), rep),
        ],
        out_specs=[
            pl.BlockSpec((_BP, _TW), row),
            pl.BlockSpec((_BP, _D), row),
        ],
        out_shape=[
            jax.ShapeDtypeStruct((npad, _TW), jnp.float32),
            jax.ShapeDtypeStruct((npad, _D), jnp.float32),
        ],
    )(xp, h1p_, c1p_, x1p_, dtp, b2, a2, W_q, b_q, W_k, b_k, W_c, b_c, U_f)


def _sc_gather(table, idx3):
    nchunk = idx3.shape[1]
    e_pad = _NW * nchunk * _CH
    mesh = plsc.VectorSubcoreMesh(core_axis_name="c", subcore_axis_name="s")

    @functools.partial(
        pl.kernel, mesh=mesh,
        out_type=jax.ShapeDtypeStruct((e_pad, _TW), jnp.float32),
        scratch_types=[
            pltpu.VMEM((nchunk, _CH), jnp.int32),
            pltpu.VMEM((_CH, _TW), jnp.float32),
            pltpu.VMEM((_CH, _TW), jnp.float32),
            pltpu.SemaphoreType.DMA,
            pltpu.SemaphoreType.DMA,
            pltpu.SemaphoreType.DMA,
            pltpu.SemaphoreType.DMA,
        ],
    )
    def k(table_hbm, idx_hbm, out_hbm, idx_v, buf0, buf1, gs0, gs1, ws0, ws1):
        wid = lax.axis_index("s") * _NC + lax.axis_index("c")
        base = wid * (nchunk * _CH)
        pltpu.sync_copy(idx_hbm.at[wid], idx_v)

        bufs = (buf0, buf1)
        gsem = (gs0, gs1)
        wsem = (ws0, ws1)

        def g_start(cc, p):
            pltpu.async_copy(table_hbm.at[idx_v.at[cc]], bufs[p], gsem[p])

        def g_wait(p):
            pltpu.make_async_copy(
                table_hbm.at[idx_v.at[0]], bufs[p], gsem[p]).wait()

        def w_start(cc, p):
            pltpu.async_copy(
                bufs[p], out_hbm.at[pl.ds(base + cc * _CH, _CH)], wsem[p])

        def w_wait(p):
            pltpu.make_async_copy(
                bufs[p], out_hbm.at[pl.ds(base, _CH)], wsem[p]).wait()

        g_start(0, 0)
        g_wait(0)
        w_start(0, 0)
        g_start(1, 1)

        def body(i, carry):
            for b in range(2):
                cc = 1 + 2 * i + b
                p = (1 + b) % 2
                g_wait(p)
                w_start(cc, p)
                w_wait(1 - p)
                g_start(cc + 1, 1 - p)
            return carry

        lax.fori_loop(0, (nchunk - 2) // 2, body, 0)

        pl1 = (nchunk - 1) % 2
        g_wait(pl1)
        w_start(nchunk - 1, pl1)
        w_wait(1 - pl1)
        w_wait(pl1)

    return k(table, idx3)


def _main_body(mail_ref, x_ref, xq_ref, f_ref, iou1_ref, mso1_ref,
               uiou_ref, umso_ref, biou_ref, bmso_ref,
               h_out_ref, c_out_ref, x_out_ref):
    glh_m = mail_ref[:, :, 0 * _D:1 * _D]
    x_m = mail_ref[:, :, 1 * _D:2 * _D]
    hk_m = mail_ref[:, :, 2 * _D:3 * _D]
    ckt_m = mail_ref[:, :, 3 * _D:4 * _D]
    tmp_m = mail_ref[:, :, 4 * _D:5 * _D]

    xq = xq_ref[...][:, None, :]
    d = 2.0 * _artanh(_norm(_mobius_add(-xq, hk_m)))
    scores = jax.nn.softmax(-d, axis=1)
    h_tild = _weighted_midpoint(_expmap0(scores * glh_m))

    f_p = _project(f_ref[...])[:, None, :]
    fg = jax.nn.sigmoid(_logmap0(
        _mobius_add(jnp.broadcast_to(f_p, tmp_m.shape), tmp_m)))
    c_red = _weighted_midpoint(_mobius_pointwise_mul(fg, ckt_m))
    x_red = jnp.mean(x_m, axis=1)

    iou1n = _mobius_add(_project(iou1_ref[...]), _mobius_matvec(uiou_ref[...], h_tild))
    mso1n = _mobius_add(_project(mso1_ref[...]), _mobius_matvec(umso_ref[...], h_tild))
    iou = _mobius_add(iou1n, biou_ref[...])
    mso = _mobius_add(mso1n, bmso_ref[...])
    i_ = jax.nn.sigmoid(_logmap0(iou[:, 0 * _D:1 * _D]))
    u_ = jnp.tanh(_logmap0(iou[:, 1 * _D:2 * _D]))
    m_ = jax.nn.sigmoid(_logmap0(mso[:, 0 * _D:1 * _D]))
    s_ = jax.nn.sigmoid(_logmap0(mso[:, 1 * _D:2 * _D]))
    o_ = jax.nn.sigmoid(_logmap0(mso[:, 2 * _D:3 * _D]))
    c_out = _mobius_add(
        _mobius_add(_mobius_pointwise_mul(i_, u_), c_red),
        _mobius_pointwise_mul(m_, s_))
    h_out_ref[...] = _mobius_pointwise_mul(o_, jnp.tanh(_logmap0(c_out)))
    c_out_ref[...] = c_out
    x_out_ref[...] = (x_red + x_ref[...]) * 0.5


def _main(mail3, xp, x_q, fp, iou1p, mso1p, U_iou, U_mso, b_iou, b_mso):
    npad = xp.shape[0]
    row = lambda i: (i, 0)
    rep = lambda i: (0, 0)
    return pl.pallas_call(
        _main_body,
        grid=(npad // _BM,),
        in_specs=[
            pl.BlockSpec((_BM, _DEG, _TW), lambda i: (i, 0, 0)),
            pl.BlockSpec((_BM, _D), row),
            pl.BlockSpec((_BM, _D), row),
            pl.BlockSpec((_BM, _D), row),
            pl.BlockSpec((_BM, 2 * _D), row),
            pl.BlockSpec((_BM, 3 * _D), row),
            pl.BlockSpec((2 * _D, _D), rep),
            pl.BlockSpec((3 * _D, _D), rep),
            pl.BlockSpec((1, 2 * _D), rep),
            pl.BlockSpec((1, 3 * _D), rep),
        ],
        out_specs=[
            pl.BlockSpec((_BM, _D), row),
            pl.BlockSpec((_BM, _D), row),
            pl.BlockSpec((_BM, _D), row),
        ],
        out_shape=[
            jax.ShapeDtypeStruct((npad, _D), jnp.float32),
            jax.ShapeDtypeStruct((npad, _D), jnp.float32),
            jax.ShapeDtypeStruct((npad, _D), jnp.float32),
        ],
    )(mail3, xp, x_q, fp, iou1p, mso1p, U_iou, U_mso, b_iou, b_mso)


_NSLICE = 8


def kernel(x, h1, c1, x1, del_t, f, iou1, mso1, W_q, b_q, W_k, b_k, W_c, b_c,
           U_f, U_iou, U_mso, b_iou, b_mso, b, a, edge_src):
    n, d = x.shape
    deg = edge_src.shape[1]
    unit = math.lcm(_BP, _NSLICE * _BM, _NSLICE * (_NW * _CH) // deg)
    npad = ((n + unit - 1) // unit) * unit
    pad = npad - n

    pad2 = lambda t: jnp.pad(t, ((0, pad), (0, 0)))
    xp = pad2(x)
    dtp = jnp.pad(del_t, (0, pad)).reshape(npad, 1)
    b2 = b.reshape(1, 1)
    a2 = a.reshape(1, 1)

    table, x_q = _prep(xp, pad2(h1), pad2(c1), pad2(x1), dtp, b2, a2,
                       W_q, b_q, W_k, b_k, W_c, b_c, U_f)

    ns = npad // _NSLICE
    nchunk = (ns * deg) // (_NW * _CH)
    idx5 = jnp.pad(edge_src.reshape(-1), (0, pad * deg)).reshape(
        _NSLICE, _NW, nchunk, _CH)
    fp, iou1p, mso1p = pad2(f), pad2(iou1), pad2(mso1)

    houts, couts, xouts = [], [], []
    for s in range(_NSLICE):
        mail3 = _sc_gather(table, idx5[s]).reshape(ns, deg, _TW)
        lo, hi = s * ns, (s + 1) * ns
        h_o, c_o, x_o = _main(mail3, xp[lo:hi], x_q[lo:hi], fp[lo:hi],
                              iou1p[lo:hi], mso1p[lo:hi],
                              U_iou, U_mso, b_iou, b_mso)
        houts.append(h_o)
        couts.append(c_o)
        xouts.append(x_o)
    h_out = jnp.concatenate(houts)
    c_out = jnp.concatenate(couts)
    x_out = jnp.concatenate(xouts)
    return h_out[:n], c_out[:n], x_out[:n]


# speedup vs baseline: 6.6126x; 1.0670x over previous
import functools
import math

import jax
import jax.numpy as jnp
from jax import lax
from jax.experimental import pallas as pl
from jax.experimental.pallas import tpu as pltpu
from jax.experimental.pallas import tpu_sc as plsc

_EPS = 1e-15
_MAXN = 1.0 - 1e-5

_D = 128
_DEG = 16
_TW = 5 * _D

_NC = 2
_NS = 16
_NW = _NC * _NS
_CH = 64

_BP = 512
_BM = 64


def _norm(x):
    return jnp.sqrt(jnp.sum(x * x, axis=-1, keepdims=True) + _EPS)


def _project(x):
    n = _norm(x)
    return x * jnp.minimum(1.0, _MAXN / n)


def _artanh(x):
    x = jnp.clip(x, -1.0 + 1e-7, 1.0 - 1e-7)
    return 0.5 * jnp.log((1.0 + x) / (1.0 - x))


def _expmap0(u):
    n = jnp.maximum(_norm(u), _EPS)
    return _project(jnp.tanh(n) * u / n)


def _logmap0(x):
    n = jnp.maximum(_norm(x), _EPS)
    return _artanh(n) * x / n


def _mobius_add(x, y):
    x2 = jnp.sum(x * x, -1, keepdims=True)
    y2 = jnp.sum(y * y, -1, keepdims=True)
    xy = jnp.sum(x * y, -1, keepdims=True)
    num = (1.0 + 2.0 * xy + y2) * x + (1.0 - x2) * y
    den = 1.0 + 2.0 * xy + x2 * y2
    return _project(num / jnp.maximum(den, _EPS))


def _mobius_matvec(M, x):
    mx = lax.dot_general(x, M, (((x.ndim - 1,), (1,)), ((), ())))
    xn = jnp.maximum(_norm(x), _EPS)
    mxn = jnp.maximum(_norm(mx), _EPS)
    return _project(jnp.tanh(mxn / xn * _artanh(xn)) * mx / mxn)


def _mobius_pointwise_mul(w, x):
    wx = w * x
    xn = jnp.maximum(_norm(x), _EPS)
    wxn = jnp.maximum(_norm(wx), _EPS)
    return _project(jnp.tanh(wxn / xn * _artanh(xn)) * wx / wxn)


def _mobius_scalar_mul(r, x):
    n = jnp.maximum(_norm(x), _EPS)
    return _project(jnp.tanh(r * _artanh(n)) * x / n)


def _weighted_midpoint(xs):
    lam = 2.0 / jnp.maximum(1.0 - jnp.sum(xs * xs, -1, keepdims=True), _EPS)
    nom = jnp.sum(lam * xs, axis=1)
    den = jnp.maximum(jnp.sum(lam - 1.0, axis=1), _EPS)
    return _mobius_scalar_mul(0.5, nom / den)


def _prep_body(x_ref, h1_ref, c1_ref, x1_ref, dt_ref, b_ref, a_ref,
               wq_ref, bq_ref, wk_ref, bk_ref, wc_ref, bc_ref, uf_ref,
               table_ref, xq_ref):
    h1p = _project(h1_ref[...])
    c1p = _project(c1_ref[...])
    xq = _mobius_add(_mobius_matvec(wq_ref[...], _expmap0(x_ref[...])), bq_ref[...])
    hk = _mobius_add(_mobius_matvec(wk_ref[...], h1p), bk_ref[...])
    c_sk = _expmap0(jnp.tanh(_logmap0(
        _mobius_add(_mobius_matvec(wc_ref[...], c1p), bc_ref[...]))))
    tmp = _mobius_matvec(uf_ref[...], h1p)
    g = b_ref[0, 0] * jnp.exp(-a_ref[0, 0] * dt_ref[...])
    csk_hat = _mobius_pointwise_mul(c_sk, g)
    ckt = _mobius_add(_mobius_add(-c_sk, c1p), csk_hat)
    table_ref[:, 0 * _D:1 * _D] = g * _logmap0(h1p)
    table_ref[:, 1 * _D:2 * _D] = x1_ref[...]
    table_ref[:, 2 * _D:3 * _D] = hk
    table_ref[:, 3 * _D:4 * _D] = _logmap0(ckt)
    table_ref[:, 4 * _D:5 * _D] = tmp
    xq_ref[...] = xq


def _prep(xp, h1p_, c1p_, x1p_, dtp, b2, a2, W_q, b_q, W_k, b_k, W_c, b_c, U_f):
    npad = xp.shape[0]
    row = lambda i: (i, 0)
    rep = lambda i: (0, 0)
    return pl.pallas_call(
        _prep_body,
        grid=(npad // _BP,),
        in_specs=[
            pl.BlockSpec((_BP, _D), row),
            pl.BlockSpec((_BP, _D), row),
            pl.BlockSpec((_BP, _D), row),
            pl.BlockSpec((_BP, _D), row),
            pl.BlockSpec((_BP, 1), row),
            pl.BlockSpec((1, 1), rep),
            pl.BlockSpec((1, 1), rep),
            pl.BlockSpec((_D, _D), rep),
            pl.BlockSpec((1, _D), rep),
            pl.BlockSpec((_D, _D), rep),
            pl.BlockSpec((1, _D), rep),
            pl.BlockSpec((_D, _D), rep),
            pl.BlockSpec((1, _D), rep),
            pl.BlockSpec((_D, _D), rep),
        ],
        out_specs=[
            pl.BlockSpec((_BP, _TW), row),
            pl.BlockSpec((_BP, _D), row),
        ],
        out_shape=[
            jax.ShapeDtypeStruct((npad, _TW), jnp.float32),
            jax.ShapeDtypeStruct((npad, _D), jnp.float32),
        ],
    )(xp, h1p_, c1p_, x1p_, dtp, b2, a2, W_q, b_q, W_k, b_k, W_c, b_c, U_f)


def _sc_gather(table, idx3):
    nchunk = idx3.shape[1]
    e_pad = _NW * nchunk * _CH
    mesh = plsc.VectorSubcoreMesh(core_axis_name="c", subcore_axis_name="s")

    @functools.partial(
        pl.kernel, mesh=mesh,
        out_type=jax.ShapeDtypeStruct((e_pad, _TW), jnp.float32),
        scratch_types=[
            pltpu.VMEM((nchunk, _CH), jnp.int32),
            pltpu.VMEM((_CH, _TW), jnp.float32),
            pltpu.VMEM((_CH, _TW), jnp.float32),
            pltpu.SemaphoreType.DMA,
            pltpu.SemaphoreType.DMA,
            pltpu.SemaphoreType.DMA,
            pltpu.SemaphoreType.DMA,
        ],
    )
    def k(table_hbm, idx_hbm, out_hbm, idx_v, buf0, buf1, gs0, gs1, ws0, ws1):
        wid = lax.axis_index("s") * _NC + lax.axis_index("c")
        base = wid * (nchunk * _CH)
        pltpu.sync_copy(idx_hbm.at[wid], idx_v)

        bufs = (buf0, buf1)
        gsem = (gs0, gs1)
        wsem = (ws0, ws1)

        def g_start(cc, p):
            pltpu.async_copy(table_hbm.at[idx_v.at[cc]], bufs[p], gsem[p])

        def g_wait(p):
            pltpu.make_async_copy(
                table_hbm.at[idx_v.at[0]], bufs[p], gsem[p]).wait()

        def w_start(cc, p):
            pltpu.async_copy(
                bufs[p], out_hbm.at[pl.ds(base + cc * _CH, _CH)], wsem[p])

        def w_wait(p):
            pltpu.make_async_copy(
                bufs[p], out_hbm.at[pl.ds(base, _CH)], wsem[p]).wait()

        g_start(0, 0)
        g_wait(0)
        w_start(0, 0)
        g_start(1, 1)

        def body(i, carry):
            for b in range(2):
                cc = 1 + 2 * i + b
                p = (1 + b) % 2
                g_wait(p)
                w_start(cc, p)
                w_wait(1 - p)
                g_start(cc + 1, 1 - p)
            return carry

        lax.fori_loop(0, (nchunk - 2) // 2, body, 0)

        pl1 = (nchunk - 1) % 2
        g_wait(pl1)
        w_start(nchunk - 1, pl1)
        w_wait(1 - pl1)
        w_wait(pl1)

    return k(table, idx3)


def _main_body(mail_ref, x_ref, xq_ref, f_ref, iou1_ref, mso1_ref,
               uiou_ref, umso_ref, biou_ref, bmso_ref,
               h_out_ref, c_out_ref, x_out_ref):
    glh_m = mail_ref[:, :, 0 * _D:1 * _D]
    x_m = mail_ref[:, :, 1 * _D:2 * _D]
    hk_m = mail_ref[:, :, 2 * _D:3 * _D]
    lckt_m = mail_ref[:, :, 3 * _D:4 * _D]
    tmp_m = mail_ref[:, :, 4 * _D:5 * _D]

    xq = xq_ref[...][:, None, :]
    x2 = jnp.sum(xq * xq, -1, keepdims=True)
    y2 = jnp.sum(hk_m * hk_m, -1, keepdims=True)
    xy = jnp.sum(xq * hk_m, -1, keepdims=True)
    a_ = 1.0 - 2.0 * xy + y2
    b_ = 1.0 - x2
    den = jnp.maximum(1.0 - 2.0 * xy + x2 * y2, _EPS)
    q2 = jnp.maximum(a_ * a_ * x2 - 2.0 * a_ * b_ * xy + b_ * b_ * y2, 0.0)
    nz = jnp.sqrt(q2 / (den * den) + _EPS)
    d = 2.0 * _artanh(jnp.minimum(nz, _MAXN))
    scores = jax.nn.softmax(-d, axis=1)
    h_tild = _weighted_midpoint(_expmap0(scores * glh_m))

    f_p = _project(f_ref[...])[:, None, :]
    fg = jax.nn.sigmoid(_logmap0(
        _mobius_add(jnp.broadcast_to(f_p, tmp_m.shape), tmp_m)))
    c_red = _weighted_midpoint(_expmap0(fg * lckt_m))
    x_red = jnp.mean(x_m, axis=1)

    iou1n = _mobius_add(_project(iou1_ref[...]), _mobius_matvec(uiou_ref[...], h_tild))
    mso1n = _mobius_add(_project(mso1_ref[...]), _mobius_matvec(umso_ref[...], h_tild))
    iou = _mobius_add(iou1n, biou_ref[...])
    mso = _mobius_add(mso1n, bmso_ref[...])
    i_ = jax.nn.sigmoid(_logmap0(iou[:, 0 * _D:1 * _D]))
    u_ = jnp.tanh(_logmap0(iou[:, 1 * _D:2 * _D]))
    m_ = jax.nn.sigmoid(_logmap0(mso[:, 0 * _D:1 * _D]))
    s_ = jax.nn.sigmoid(_logmap0(mso[:, 1 * _D:2 * _D]))
    o_ = jax.nn.sigmoid(_logmap0(mso[:, 2 * _D:3 * _D]))
    c_out = _mobius_add(
        _mobius_add(_mobius_pointwise_mul(i_, u_), c_red),
        _mobius_pointwise_mul(m_, s_))
    h_out_ref[...] = _mobius_pointwise_mul(o_, jnp.tanh(_logmap0(c_out)))
    c_out_ref[...] = c_out
    x_out_ref[...] = (x_red + x_ref[...]) * 0.5


def _main(mail3, xp, x_q, fp, iou1p, mso1p, U_iou, U_mso, b_iou, b_mso):
    npad = xp.shape[0]
    row = lambda i: (i, 0)
    rep = lambda i: (0, 0)
    return pl.pallas_call(
        _main_body,
        grid=(npad // _BM,),
        in_specs=[
            pl.BlockSpec((_BM, _DEG, _TW), lambda i: (i, 0, 0)),
            pl.BlockSpec((_BM, _D), row),
            pl.BlockSpec((_BM, _D), row),
            pl.BlockSpec((_BM, _D), row),
            pl.BlockSpec((_BM, 2 * _D), row),
            pl.BlockSpec((_BM, 3 * _D), row),
            pl.BlockSpec((2 * _D, _D), rep),
            pl.BlockSpec((3 * _D, _D), rep),
            pl.BlockSpec((1, 2 * _D), rep),
            pl.BlockSpec((1, 3 * _D), rep),
        ],
        out_specs=[
            pl.BlockSpec((_BM, _D), row),
            pl.BlockSpec((_BM, _D), row),
            pl.BlockSpec((_BM, _D), row),
        ],
        out_shape=[
            jax.ShapeDtypeStruct((npad, _D), jnp.float32),
            jax.ShapeDtypeStruct((npad, _D), jnp.float32),
            jax.ShapeDtypeStruct((npad, _D), jnp.float32),
        ],
    )(mail3, xp, x_q, fp, iou1p, mso1p, U_iou, U_mso, b_iou, b_mso)


_NSLICE = 8


def kernel(x, h1, c1, x1, del_t, f, iou1, mso1, W_q, b_q, W_k, b_k, W_c, b_c,
           U_f, U_iou, U_mso, b_iou, b_mso, b, a, edge_src):
    n, d = x.shape
    deg = edge_src.shape[1]
    unit = math.lcm(_BP, _NSLICE * _BM, _NSLICE * (_NW * _CH) // deg)
    npad = ((n + unit - 1) // unit) * unit
    pad = npad - n

    pad2 = lambda t: jnp.pad(t, ((0, pad), (0, 0)))
    xp = pad2(x)
    dtp = jnp.pad(del_t, (0, pad)).reshape(npad, 1)
    b2 = b.reshape(1, 1)
    a2 = a.reshape(1, 1)

    table, x_q = _prep(xp, pad2(h1), pad2(c1), pad2(x1), dtp, b2, a2,
                       W_q, b_q, W_k, b_k, W_c, b_c, U_f)

    ns = npad // _NSLICE
    nchunk = (ns * deg) // (_NW * _CH)
    idx5 = jnp.pad(edge_src.reshape(-1), (0, pad * deg)).reshape(
        _NSLICE, _NW, nchunk, _CH)
    fp, iou1p, mso1p = pad2(f), pad2(iou1), pad2(mso1)

    houts, couts, xouts = [], [], []
    for s in range(_NSLICE):
        mail3 = _sc_gather(table, idx5[s]).reshape(ns, deg, _TW)
        lo, hi = s * ns, (s + 1) * ns
        h_o, c_o, x_o = _main(mail3, xp[lo:hi], x_q[lo:hi], fp[lo:hi],
                              iou1p[lo:hi], mso1p[lo:hi],
                              U_iou, U_mso, b_iou, b_mso)
        houts.append(h_o)
        couts.append(c_o)
        xouts.append(x_o)
    h_out = jnp.concatenate(houts)
    c_out = jnp.concatenate(couts)
    x_out = jnp.concatenate(xouts)
    return h_out[:n], c_out[:n], x_out[:n]


# speedup vs baseline: 6.6365x; 1.0036x over previous
import functools
import math

import jax
import jax.numpy as jnp
from jax import lax
from jax.experimental import pallas as pl
from jax.experimental.pallas import tpu as pltpu
from jax.experimental.pallas import tpu_sc as plsc

_EPS = 1e-15
_MAXN = 1.0 - 1e-5

_D = 128
_DEG = 16
_TW = 5 * _D

_NC = 2
_NS = 16
_NW = _NC * _NS
_CH = 64

_BP = 512
_BM = 64


def _norm(x):
    return jnp.sqrt(jnp.sum(x * x, axis=-1, keepdims=True) + _EPS)


def _project(x):
    n = _norm(x)
    return x * jnp.minimum(1.0, _MAXN / n)


def _artanh(x):
    x = jnp.clip(x, -1.0 + 1e-7, 1.0 - 1e-7)
    return 0.5 * jnp.log((1.0 + x) / (1.0 - x))


def _expmap0(u):
    n = jnp.maximum(_norm(u), _EPS)
    return _project(jnp.tanh(n) * u / n)


def _logmap0(x):
    n = jnp.maximum(_norm(x), _EPS)
    return _artanh(n) * x / n


def _mobius_add(x, y):
    x2 = jnp.sum(x * x, -1, keepdims=True)
    y2 = jnp.sum(y * y, -1, keepdims=True)
    xy = jnp.sum(x * y, -1, keepdims=True)
    num = (1.0 + 2.0 * xy + y2) * x + (1.0 - x2) * y
    den = 1.0 + 2.0 * xy + x2 * y2
    return _project(num / jnp.maximum(den, _EPS))


def _mobius_matvec(M, x):
    mx = lax.dot_general(x, M, (((x.ndim - 1,), (1,)), ((), ())))
    xn = jnp.maximum(_norm(x), _EPS)
    mxn = jnp.maximum(_norm(mx), _EPS)
    return _project(jnp.tanh(mxn / xn * _artanh(xn)) * mx / mxn)


def _mobius_pointwise_mul(w, x):
    wx = w * x
    xn = jnp.maximum(_norm(x), _EPS)
    wxn = jnp.maximum(_norm(wx), _EPS)
    return _project(jnp.tanh(wxn / xn * _artanh(xn)) * wx / wxn)


def _mobius_scalar_mul(r, x):
    n = jnp.maximum(_norm(x), _EPS)
    return _project(jnp.tanh(r * _artanh(n)) * x / n)


def _rnorm3(x):
    return jnp.sqrt(jnp.sum(x * x, axis=-1) + _EPS)


def _project3(x):
    n = _rnorm3(x)
    return x * jnp.minimum(1.0, _MAXN / n)[..., None]


def _expmap03(u):
    n = jnp.maximum(_rnorm3(u), _EPS)
    return _project3(u * (jnp.tanh(n) / n)[..., None])


def _logmap03(x):
    n = jnp.maximum(_rnorm3(x), _EPS)
    return x * (_artanh(n) / n)[..., None]


def _mobius_add3(x, y):
    x2 = jnp.sum(x * x, -1)
    y2 = jnp.sum(y * y, -1)
    xy = jnp.sum(x * y, -1)
    num = (1.0 + 2.0 * xy + y2)[..., None] * x + (1.0 - x2)[..., None] * y
    den = 1.0 + 2.0 * xy + x2 * y2
    return _project3(num / jnp.maximum(den, _EPS)[..., None])


def _weighted_midpoint3(xs):
    lam = 2.0 / jnp.maximum(1.0 - jnp.sum(xs * xs, -1), _EPS)
    nom = jnp.sum(lam[..., None] * xs, axis=1)
    den = jnp.maximum(jnp.sum(lam - 1.0, axis=1, keepdims=True), _EPS)
    return _mobius_scalar_mul(0.5, nom / den)


def _prep_body(x_ref, h1_ref, c1_ref, x1_ref, dt_ref, b_ref, a_ref,
               wq_ref, bq_ref, wk_ref, bk_ref, wc_ref, bc_ref, uf_ref,
               table_ref, xq_ref):
    h1p = _project(h1_ref[...])
    c1p = _project(c1_ref[...])
    xq = _mobius_add(_mobius_matvec(wq_ref[...], _expmap0(x_ref[...])), bq_ref[...])
    hk = _mobius_add(_mobius_matvec(wk_ref[...], h1p), bk_ref[...])
    c_sk = _expmap0(jnp.tanh(_logmap0(
        _mobius_add(_mobius_matvec(wc_ref[...], c1p), bc_ref[...]))))
    tmp = _mobius_matvec(uf_ref[...], h1p)
    g = b_ref[0, 0] * jnp.exp(-a_ref[0, 0] * dt_ref[...])
    csk_hat = _mobius_pointwise_mul(c_sk, g)
    ckt = _mobius_add(_mobius_add(-c_sk, c1p), csk_hat)
    table_ref[:, 0 * _D:1 * _D] = g * _logmap0(h1p)
    table_ref[:, 1 * _D:2 * _D] = x1_ref[...]
    table_ref[:, 2 * _D:3 * _D] = hk
    table_ref[:, 3 * _D:4 * _D] = _logmap0(ckt)
    table_ref[:, 4 * _D:5 * _D] = tmp
    xq_ref[...] = xq


def _prep(xp, h1p_, c1p_, x1p_, dtp, b2, a2, W_q, b_q, W_k, b_k, W_c, b_c, U_f):
    npad = xp.shape[0]
    row = lambda i: (i, 0)
    rep = lambda i: (0, 0)
    return pl.pallas_call(
        _prep_body,
        grid=(npad // _BP,),
        in_specs=[
            pl.BlockSpec((_BP, _D), row),
            pl.BlockSpec((_BP, _D), row),
            pl.BlockSpec((_BP, _D), row),
            pl.BlockSpec((_BP, _D), row),
            pl.BlockSpec((_BP, 1), row),
            pl.BlockSpec((1, 1), rep),
            pl.BlockSpec((1, 1), rep),
            pl.BlockSpec((_D, _D), rep),
            pl.BlockSpec((1, _D), rep),
            pl.BlockSpec((_D, _D), rep),
            pl.BlockSpec((1, _D), rep),
            pl.BlockSpec((_D, _D), rep),
            pl.BlockSpec((1, _D), rep),
            pl.BlockSpec((_D, _D), rep),
        ],
        out_specs=[
            pl.BlockSpec((_BP, _TW), row),
            pl.BlockSpec((_BP, _D), row),
        ],
        out_shape=[
            jax.ShapeDtypeStruct((npad, _TW), jnp.float32),
            jax.ShapeDtypeStruct((npad, _D), jnp.float32),
        ],
    )(xp, h1p_, c1p_, x1p_, dtp, b2, a2, W_q, b_q, W_k, b_k, W_c, b_c, U_f)


def _sc_gather(table, idx3):
    nchunk = idx3.shape[1]
    e_pad = _NW * nchunk * _CH
    mesh = plsc.VectorSubcoreMesh(core_axis_name="c", subcore_axis_name="s")

    @functools.partial(
        pl.kernel, mesh=mesh,
        out_type=jax.ShapeDtypeStruct((e_pad, _TW), jnp.float32),
        scratch_types=[
            pltpu.VMEM((nchunk, _CH), jnp.int32),
            pltpu.VMEM((_CH, _TW), jnp.float32),
            pltpu.VMEM((_CH, _TW), jnp.float32),
            pltpu.SemaphoreType.DMA,
            pltpu.SemaphoreType.DMA,
            pltpu.SemaphoreType.DMA,
            pltpu.SemaphoreType.DMA,
        ],
    )
    def k(table_hbm, idx_hbm, out_hbm, idx_v, buf0, buf1, gs0, gs1, ws0, ws1):
        wid = lax.axis_index("s") * _NC + lax.axis_index("c")
        base = wid * (nchunk * _CH)
        pltpu.sync_copy(idx_hbm.at[wid], idx_v)

        bufs = (buf0, buf1)
        gsem = (gs0, gs1)
        wsem = (ws0, ws1)

        def g_start(cc, p):
            pltpu.async_copy(table_hbm.at[idx_v.at[cc]], bufs[p], gsem[p])

        def g_wait(p):
            pltpu.make_async_copy(
                table_hbm.at[idx_v.at[0]], bufs[p], gsem[p]).wait()

        def w_start(cc, p):
            pltpu.async_copy(
                bufs[p], out_hbm.at[pl.ds(base + cc * _CH, _CH)], wsem[p])

        def w_wait(p):
            pltpu.make_async_copy(
                bufs[p], out_hbm.at[pl.ds(base, _CH)], wsem[p]).wait()

        g_start(0, 0)
        g_wait(0)
        w_start(0, 0)
        g_start(1, 1)

        def body(i, carry):
            for b in range(2):
                cc = 1 + 2 * i + b
                p = (1 + b) % 2
                g_wait(p)
                w_start(cc, p)
                w_wait(1 - p)
                g_start(cc + 1, 1 - p)
            return carry

        lax.fori_loop(0, (nchunk - 2) // 2, body, 0)

        pl1 = (nchunk - 1) % 2
        g_wait(pl1)
        w_start(nchunk - 1, pl1)
        w_wait(1 - pl1)
        w_wait(pl1)

    return k(table, idx3)


def _main_body(mail_ref, x_ref, xq_ref, f_ref, iou1_ref, mso1_ref,
               uiou_ref, umso_ref, biou_ref, bmso_ref,
               h_out_ref, c_out_ref, x_out_ref):
    glh_m = mail_ref[:, :, 0 * _D:1 * _D]
    x_m = mail_ref[:, :, 1 * _D:2 * _D]
    hk_m = mail_ref[:, :, 2 * _D:3 * _D]
    lckt_m = mail_ref[:, :, 3 * _D:4 * _D]
    tmp_m = mail_ref[:, :, 4 * _D:5 * _D]

    xq = xq_ref[...]
    x2 = jnp.sum(xq * xq, -1, keepdims=True)
    y2 = jnp.sum(hk_m * hk_m, -1)
    xy = jnp.sum(xq[:, None, :] * hk_m, -1)
    a_ = 1.0 - 2.0 * xy + y2
    b_ = 1.0 - x2
    den = jnp.maximum(1.0 - 2.0 * xy + x2 * y2, _EPS)
    q2 = jnp.maximum(a_ * a_ * x2 - 2.0 * a_ * b_ * xy + b_ * b_ * y2, 0.0)
    nz = jnp.sqrt(q2 / (den * den) + _EPS)
    z = jnp.minimum(nz, _MAXN)
    w = (1.0 - z) / (1.0 + z)
    scores = w / jnp.sum(w, axis=1, keepdims=True)
    h_tild = _weighted_midpoint3(_expmap03(scores[..., None] * glh_m))

    f_p = _project(f_ref[...])[:, None, :]
    fg = jax.nn.sigmoid(_logmap03(
        _mobius_add3(jnp.broadcast_to(f_p, tmp_m.shape), tmp_m)))
    c_red = _weighted_midpoint3(_expmap03(fg * lckt_m))
    x_red = jnp.mean(x_m, axis=1)

    iou1n = _mobius_add(_project(iou1_ref[...]), _mobius_matvec(uiou_ref[...], h_tild))
    mso1n = _mobius_add(_project(mso1_ref[...]), _mobius_matvec(umso_ref[...], h_tild))
    iou = _mobius_add(iou1n, biou_ref[...])
    mso = _mobius_add(mso1n, bmso_ref[...])
    i_ = jax.nn.sigmoid(_logmap0(iou[:, 0 * _D:1 * _D]))
    u_ = jnp.tanh(_logmap0(iou[:, 1 * _D:2 * _D]))
    m_ = jax.nn.sigmoid(_logmap0(mso[:, 0 * _D:1 * _D]))
    s_ = jax.nn.sigmoid(_logmap0(mso[:, 1 * _D:2 * _D]))
    o_ = jax.nn.sigmoid(_logmap0(mso[:, 2 * _D:3 * _D]))
    c_out = _mobius_add(
        _mobius_add(_mobius_pointwise_mul(i_, u_), c_red),
        _mobius_pointwise_mul(m_, s_))
    h_out_ref[...] = _mobius_pointwise_mul(o_, jnp.tanh(_logmap0(c_out)))
    c_out_ref[...] = c_out
    x_out_ref[...] = (x_red + x_ref[...]) * 0.5


def _main(mail3, xp, x_q, fp, iou1p, mso1p, U_iou, U_mso, b_iou, b_mso):
    npad = xp.shape[0]
    row = lambda i: (i, 0)
    rep = lambda i: (0, 0)
    return pl.pallas_call(
        _main_body,
        grid=(npad // _BM,),
        in_specs=[
            pl.BlockSpec((_BM, _DEG, _TW), lambda i: (i, 0, 0)),
            pl.BlockSpec((_BM, _D), row),
            pl.BlockSpec((_BM, _D), row),
            pl.BlockSpec((_BM, _D), row),
            pl.BlockSpec((_BM, 2 * _D), row),
            pl.BlockSpec((_BM, 3 * _D), row),
            pl.BlockSpec((2 * _D, _D), rep),
            pl.BlockSpec((3 * _D, _D), rep),
            pl.BlockSpec((1, 2 * _D), rep),
            pl.BlockSpec((1, 3 * _D), rep),
        ],
        out_specs=[
            pl.BlockSpec((_BM, _D), row),
            pl.BlockSpec((_BM, _D), row),
            pl.BlockSpec((_BM, _D), row),
        ],
        out_shape=[
            jax.ShapeDtypeStruct((npad, _D), jnp.float32),
            jax.ShapeDtypeStruct((npad, _D), jnp.float32),
            jax.ShapeDtypeStruct((npad, _D), jnp.float32),
        ],
    )(mail3, xp, x_q, fp, iou1p, mso1p, U_iou, U_mso, b_iou, b_mso)


_NSLICE = 8


def kernel(x, h1, c1, x1, del_t, f, iou1, mso1, W_q, b_q, W_k, b_k, W_c, b_c,
           U_f, U_iou, U_mso, b_iou, b_mso, b, a, edge_src):
    n, d = x.shape
    deg = edge_src.shape[1]
    unit = math.lcm(_BP, _NSLICE * _BM, _NSLICE * (_NW * _CH) // deg)
    npad = ((n + unit - 1) // unit) * unit
    pad = npad - n

    pad2 = lambda t: jnp.pad(t, ((0, pad), (0, 0)))
    xp = pad2(x)
    dtp = jnp.pad(del_t, (0, pad)).reshape(npad, 1)
    b2 = b.reshape(1, 1)
    a2 = a.reshape(1, 1)

    table, x_q = _prep(xp, pad2(h1), pad2(c1), pad2(x1), dtp, b2, a2,
                       W_q, b_q, W_k, b_k, W_c, b_c, U_f)

    ns = npad // _NSLICE
    nchunk = (ns * deg) // (_NW * _CH)
    idx5 = jnp.pad(edge_src.reshape(-1), (0, pad * deg)).reshape(
        _NSLICE, _NW, nchunk, _CH)
    fp, iou1p, mso1p = pad2(f), pad2(iou1), pad2(mso1)

    houts, couts, xouts = [], [], []
    for s in range(_NSLICE):
        mail3 = _sc_gather(table, idx5[s]).reshape(ns, deg, _TW)
        lo, hi = s * ns, (s + 1) * ns
        h_o, c_o, x_o = _main(mail3, xp[lo:hi], x_q[lo:hi], fp[lo:hi],
                              iou1p[lo:hi], mso1p[lo:hi],
                              U_iou, U_mso, b_iou, b_mso)
        houts.append(h_o)
        couts.append(c_o)
        xouts.append(x_o)
    h_out = jnp.concatenate(houts)
    c_out = jnp.concatenate(couts)
    x_out = jnp.concatenate(xouts)
    return h_out[:n], c_out[:n], x_out[:n]
